# baseline (speedup 1.0000x reference)
import functools
import math

import jax
import jax.numpy as jnp
from jax import lax
from jax.experimental import pallas as pl
from jax.experimental.pallas import tpu as pltpu
from jax.experimental.pallas import tpu_sc as plsc

_D = 1024
_LANES = 16
_NC = 2
_NS = 16
_NW = _NC * _NS
_SCALE = math.sqrt(_D)
_CHUNK = 16


def _emb_body(x_hbm, wtab_hbm, ptab_hbm, out_hbm,
              idx_all, w0, w1, w2, pb0, pb1, ob0, ob1,
              g0, g1, g2, q0, q1, o0, o1,
              *, batch, seq):
    wid = lax.axis_index("s") * _NC + lax.axis_index("c")
    pos_per_w = seq // _NW
    kmax = pos_per_w // _CHUNK
    n_chunks = kmax * batch
    jcols = _D // _LANES
    xrows_per_b = seq // _CHUNK

    w = [w0, w1, w2]
    pb = [pb0, pb1]
    ob = [ob0, ob1]
    gsem = [g0, g1, g2]
    psem = [q0, q1]
    osem = [o0, o1]

    for b in range(batch):
        pltpu.sync_copy(x_hbm.at[pl.ds(b * xrows_per_b + wid * kmax, kmax)],
                        idx_all.at[pl.ds(b * kmax, kmax)])

    def start_gather(cc):
        k, b = cc // batch, cc % batch
        return pltpu.async_copy(
            wtab_hbm.at[idx_all.at[b * kmax + k]], w[cc % 3], gsem[cc % 3])

    def start_pos(k):
        return pltpu.async_copy(
            ptab_hbm.at[pl.ds(wid * pos_per_w + k * _CHUNK, _CHUNK)],
            pb[k % 2], psem[k % 2])

    gdesc = [None] * n_chunks
    pdesc = [None] * kmax
    odesc = [None] * n_chunks
    pdesc[0] = start_pos(0)
    gdesc[0] = start_gather(0)
    if kmax > 1:
        pdesc[1] = start_pos(1)
    if n_chunks > 1:
        gdesc[1] = start_gather(1)

    for cc in range(n_chunks):
        k, b = cc // batch, cc % batch
        if cc + 2 < n_chunks:
            gdesc[cc + 2] = start_gather(cc + 2)
        if b == 0 and k >= 1 and k + 1 < kmax:
            pdesc[k + 1] = start_pos(k + 1)
        gdesc[cc].wait()
        if b == 0:
            pdesc[k].wait()
        if cc - 2 >= 0:
            odesc[cc - 2].wait()

        def do_row(i, _, wb=w[cc % 3], pbk=pb[k % 2], obc=ob[cc % 2]):
            @plsc.parallel_loop(0, jcols, unroll=8)
            def do_j(j):
                sl = pl.ds(j * _LANES, _LANES)
                obc[i, sl] = wb[i, sl] * _SCALE + pbk[i, sl]
            return 0

        lax.fori_loop(0, _CHUNK, do_row, 0)

        row0 = b * seq + wid * pos_per_w + k * _CHUNK
        odesc[cc] = pltpu.async_copy(
            ob[cc % 2], out_hbm.at[pl.ds(row0, _CHUNK)], osem[cc % 2])

    for cc in range(max(0, n_chunks - 2), n_chunks):
        odesc[cc].wait()


def kernel(x, word_emb, pos_emb):
    batch, seq = x.shape
    nrows = batch * seq

    mesh = plsc.VectorSubcoreMesh(core_axis_name="c", subcore_axis_name="s")
    body = functools.partial(_emb_body, batch=batch, seq=seq)
    out = pl.kernel(
        body,
        out_type=jax.ShapeDtypeStruct((nrows, _D), jnp.float32),
        mesh=mesh,
        scratch_types=[
            pltpu.VMEM((batch * (seq // _NW // _CHUNK), _CHUNK), jnp.int32),
            pltpu.VMEM((_CHUNK, _D), jnp.float32),
            pltpu.VMEM((_CHUNK, _D), jnp.float32),
            pltpu.VMEM((_CHUNK, _D), jnp.float32),
            pltpu.VMEM((_CHUNK, _D), jnp.float32),
            pltpu.VMEM((_CHUNK, _D), jnp.float32),
            pltpu.VMEM((_CHUNK, _D), jnp.float32),
            pltpu.VMEM((_CHUNK, _D), jnp.float32),
        ] + [pltpu.SemaphoreType.DMA] * 7,
    )(x.reshape(nrows // _CHUNK, _CHUNK).astype(jnp.int32), word_emb, pos_emb)
    return out.reshape(batch, seq, _D)


# speedup vs baseline: 1.0633x; 1.0633x over previous
import functools
import math

import jax
import jax.numpy as jnp
from jax import lax
from jax.experimental import pallas as pl
from jax.experimental.pallas import tpu as pltpu
from jax.experimental.pallas import tpu_sc as plsc

_D = 1024
_LANES = 16
_NC = 2
_NS = 16
_NW = _NC * _NS
_SCALE = math.sqrt(_D)
_CHUNK = 16


def _emb_body(x_hbm, wtab_hbm, ptab_hbm, out_hbm,
              idx_all, w0, w1, w2, pb0, pb1, ob0, ob1,
              g0, g1, g2, q0, q1, o0, o1,
              *, batch, seq):
    wid = lax.axis_index("s") * _NC + lax.axis_index("c")
    pos_per_w = seq // _NW
    kmax = pos_per_w // _CHUNK
    n_chunks = kmax * batch
    jcols = _D // _LANES
    xrows_per_b = seq // _CHUNK

    w = [w0, w1, w2]
    pb = [pb0, pb1]
    ob = [ob0, ob1]
    gsem = [g0, g1, g2]
    psem = [q0, q1]
    osem = [o0, o1]

    def stage_idx(b):
        pltpu.sync_copy(x_hbm.at[pl.ds(b * xrows_per_b + wid * kmax, kmax)],
                        idx_all.at[pl.ds(b * kmax, kmax)])

    def start_gather(cc):
        k, b = cc // batch, cc % batch
        return pltpu.async_copy(
            wtab_hbm.at[idx_all.at[b * kmax + k]], w[cc % 3], gsem[cc % 3])

    def start_pos(k):
        return pltpu.async_copy(
            ptab_hbm.at[pl.ds(wid * pos_per_w + k * _CHUNK, _CHUNK)],
            pb[k % 2], psem[k % 2])

    gdesc = [None] * n_chunks
    pdesc = [None] * kmax
    odesc = [None] * n_chunks
    stage_idx(0)
    pdesc[0] = start_pos(0)
    gdesc[0] = start_gather(0)
    if n_chunks > 1:
        stage_idx(1)
        gdesc[1] = start_gather(1)
    for b in range(2, batch):
        stage_idx(b)
    if kmax > 1:
        pdesc[1] = start_pos(1)

    for cc in range(n_chunks):
        k, b = cc // batch, cc % batch
        if cc + 2 < n_chunks:
            gdesc[cc + 2] = start_gather(cc + 2)
        if b == 0 and k >= 1 and k + 1 < kmax:
            pdesc[k + 1] = start_pos(k + 1)
        gdesc[cc].wait()
        if b == 0:
            pdesc[k].wait()
        if cc - 2 >= 0:
            odesc[cc - 2].wait()

        def do_row(i, _, wb=w[cc % 3], pbk=pb[k % 2], obc=ob[cc % 2]):
            @plsc.parallel_loop(0, jcols, unroll=4)
            def do_j(j):
                sl = pl.ds(j * _LANES, _LANES)
                obc[i, sl] = wb[i, sl] * _SCALE + pbk[i, sl]
            return 0

        lax.fori_loop(0, _CHUNK, do_row, 0)

        row0 = b * seq + wid * pos_per_w + k * _CHUNK
        odesc[cc] = pltpu.async_copy(
            ob[cc % 2], out_hbm.at[pl.ds(row0, _CHUNK)], osem[cc % 2])

    for cc in range(max(0, n_chunks - 2), n_chunks):
        odesc[cc].wait()


def kernel(x, word_emb, pos_emb):
    batch, seq = x.shape
    nrows = batch * seq

    mesh = plsc.VectorSubcoreMesh(core_axis_name="c", subcore_axis_name="s")
    body = functools.partial(_emb_body, batch=batch, seq=seq)
    out = pl.kernel(
        body,
        out_type=jax.ShapeDtypeStruct((nrows, _D), jnp.float32),
        mesh=mesh,
        scratch_types=[
            pltpu.VMEM((batch * (seq // _NW // _CHUNK), _CHUNK), jnp.int32),
            pltpu.VMEM((_CHUNK, _D), jnp.float32),
            pltpu.VMEM((_CHUNK, _D), jnp.float32),
            pltpu.VMEM((_CHUNK, _D), jnp.float32),
            pltpu.VMEM((_CHUNK, _D), jnp.float32),
            pltpu.VMEM((_CHUNK, _D), jnp.float32),
            pltpu.VMEM((_CHUNK, _D), jnp.float32),
            pltpu.VMEM((_CHUNK, _D), jnp.float32),
        ] + [pltpu.SemaphoreType.DMA] * 7,
    )(x.reshape(nrows // _CHUNK, _CHUNK).astype(jnp.int32), word_emb, pos_emb)
    return out.reshape(batch, seq, _D)
